# baseline (device time: 104865 ns/iter reference)
import jax
import jax.numpy as jnp
from jax import lax
from jax.experimental import pallas as pl
from jax.experimental.pallas import tpu as pltpu

N_Y = 4


def kernel(dy, W):
    m, k = dy.shape
    n = W.shape[0]
    ch = m // N_Y

    def body(dy_ref, w_ref, out_ref, acc_ref, send_ref, rs_recv_ref,
             rs_send_sems, rs_recv_sems, ag_send_sems, ag_recv_sems):
        my_x = lax.axis_index("x")
        my_y = lax.axis_index("y")
        my_z = lax.axis_index("z")
        right = (my_y + 1) % N_Y
        left = (my_y + N_Y - 1) % N_Y

        barrier = pltpu.get_barrier_semaphore()
        for nbr in (left, right):
            pl.semaphore_signal(
                barrier, inc=1,
                device_id=(my_x, nbr, my_z),
                device_id_type=pl.DeviceIdType.MESH,
            )
        pl.semaphore_wait(barrier, 2)

        acc_ref[...] = lax.dot_general(
            dy_ref[...], w_ref[...],
            (((1,), (1,)), ((), ())),
            preferred_element_type=jnp.float32,
        )

        for s in range(N_Y - 1):
            c = (my_y + N_Y - s) % N_Y
            rows = pl.ds(c * ch, ch)
            if s == 0:
                send_ref[s] = acc_ref[rows, :]
            else:
                send_ref[s] = acc_ref[rows, :] + rs_recv_ref[s - 1]
            rdma = pltpu.make_async_remote_copy(
                src_ref=send_ref.at[s],
                dst_ref=rs_recv_ref.at[s],
                send_sem=rs_send_sems.at[s],
                recv_sem=rs_recv_sems.at[s],
                device_id=(my_x, right, my_z),
                device_id_type=pl.DeviceIdType.MESH,
            )
            rdma.start()
            rdma.wait()

        c_own = (my_y + 1) % N_Y
        own_rows = pl.ds(c_own * ch, ch)
        out_ref[own_rows, :] = acc_ref[own_rows, :] + rs_recv_ref[N_Y - 2]

        for s in range(N_Y - 1):
            c = (my_y + 1 + N_Y - s) % N_Y
            rows = pl.ds(c * ch, ch)
            rdma = pltpu.make_async_remote_copy(
                src_ref=out_ref.at[rows, :],
                dst_ref=out_ref.at[rows, :],
                send_sem=ag_send_sems.at[s],
                recv_sem=ag_recv_sems.at[s],
                device_id=(my_x, right, my_z),
                device_id_type=pl.DeviceIdType.MESH,
            )
            rdma.start()
            rdma.wait()

    return pl.pallas_call(
        body,
        out_shape=jax.ShapeDtypeStruct((m, n), jnp.float32),
        in_specs=[
            pl.BlockSpec(memory_space=pltpu.VMEM),
            pl.BlockSpec(memory_space=pltpu.VMEM),
        ],
        out_specs=pl.BlockSpec(memory_space=pltpu.VMEM),
        scratch_shapes=[
            pltpu.VMEM((m, n), jnp.float32),
            pltpu.VMEM((N_Y - 1, ch, n), jnp.float32),
            pltpu.VMEM((N_Y - 1, ch, n), jnp.float32),
            pltpu.SemaphoreType.DMA((N_Y - 1,)),
            pltpu.SemaphoreType.DMA((N_Y - 1,)),
            pltpu.SemaphoreType.DMA((N_Y - 1,)),
            pltpu.SemaphoreType.DMA((N_Y - 1,)),
        ],
        compiler_params=pltpu.CompilerParams(collective_id=0),
    )(dy, W)


# device time: 75959 ns/iter; 1.3805x vs baseline; 1.3805x over previous
import jax
import jax.numpy as jnp
from jax import lax
from jax.experimental import pallas as pl
from jax.experimental.pallas import tpu as pltpu

N_Y = 4


def kernel(dy, W):
    m, k = dy.shape
    n = W.shape[0]
    half = m // 2
    ch = half // N_Y

    def body(dy_ref, w_ref, out_ref, acc_ref, send_ref, rs_recv_ref,
             rs_send_sems, rs_recv_sems, ag_send_sems, ag_recv_sems,
             x_send_sems, x_recv_sems):
        my_x = lax.axis_index("x")
        my_y = lax.axis_index("y")
        my_z = lax.axis_index("z")
        right = (my_y + 1) % N_Y
        left = (my_y + N_Y - 1) % N_Y
        other_x = 1 - my_x
        half_off = my_x * half

        barrier = pltpu.get_barrier_semaphore()
        for dev in ((my_x, left, my_z), (my_x, right, my_z),
                    (other_x, my_y, my_z)):
            pl.semaphore_signal(
                barrier, inc=1,
                device_id=dev, device_id_type=pl.DeviceIdType.MESH,
            )
        pl.semaphore_wait(barrier, 3)

        acc_ref[...] = lax.dot_general(
            dy_ref[pl.ds(half_off, half), :], w_ref[...],
            (((1,), (1,)), ((), ())),
            preferred_element_type=jnp.float32,
        )

        for s in range(N_Y - 1):
            c = (my_y + N_Y - s) % N_Y
            rows = pl.ds(c * ch, ch)
            if s == 0:
                send_ref[s] = acc_ref[rows, :]
            else:
                send_ref[s] = acc_ref[rows, :] + rs_recv_ref[s - 1]
            rdma = pltpu.make_async_remote_copy(
                src_ref=send_ref.at[s],
                dst_ref=rs_recv_ref.at[s],
                send_sem=rs_send_sems.at[s],
                recv_sem=rs_recv_sems.at[s],
                device_id=(my_x, right, my_z),
                device_id_type=pl.DeviceIdType.MESH,
            )
            rdma.start()
            rdma.wait()

        c_own = (my_y + 1) % N_Y
        own_rows = pl.ds(half_off + c_own * ch, ch)
        out_ref[own_rows, :] = (
            acc_ref[pl.ds(c_own * ch, ch), :] + rs_recv_ref[N_Y - 2]
        )

        x_sends = []
        x0 = pltpu.make_async_remote_copy(
            src_ref=out_ref.at[own_rows, :],
            dst_ref=out_ref.at[own_rows, :],
            send_sem=x_send_sems.at[0],
            recv_sem=x_recv_sems.at[0],
            device_id=(other_x, my_y, my_z),
            device_id_type=pl.DeviceIdType.MESH,
        )
        x0.start()
        x_sends.append(x0)

        for s in range(N_Y - 1):
            c = (my_y + 1 + N_Y - s) % N_Y
            rows = pl.ds(half_off + c * ch, ch)
            rdma = pltpu.make_async_remote_copy(
                src_ref=out_ref.at[rows, :],
                dst_ref=out_ref.at[rows, :],
                send_sem=ag_send_sems.at[s],
                recv_sem=ag_recv_sems.at[s],
                device_id=(my_x, right, my_z),
                device_id_type=pl.DeviceIdType.MESH,
            )
            rdma.start()
            rdma.wait()
            c_new = (my_y + N_Y - s) % N_Y
            new_rows = pl.ds(half_off + c_new * ch, ch)
            xs = pltpu.make_async_remote_copy(
                src_ref=out_ref.at[new_rows, :],
                dst_ref=out_ref.at[new_rows, :],
                send_sem=x_send_sems.at[s + 1],
                recv_sem=x_recv_sems.at[s + 1],
                device_id=(other_x, my_y, my_z),
                device_id_type=pl.DeviceIdType.MESH,
            )
            xs.start()
            x_sends.append(xs)

        other_off = other_x * half
        for j in range(N_Y):
            c_j = c_own if j == 0 else (my_y + N_Y - (j - 1)) % N_Y
            rows_p = pl.ds(other_off + c_j * ch, ch)
            recv = pltpu.make_async_remote_copy(
                src_ref=out_ref.at[rows_p, :],
                dst_ref=out_ref.at[rows_p, :],
                send_sem=x_send_sems.at[j],
                recv_sem=x_recv_sems.at[j],
                device_id=(other_x, my_y, my_z),
                device_id_type=pl.DeviceIdType.MESH,
            )
            recv.wait_recv()
        for d in x_sends:
            d.wait_send()

    return pl.pallas_call(
        body,
        out_shape=jax.ShapeDtypeStruct((m, n), jnp.float32),
        in_specs=[
            pl.BlockSpec(memory_space=pltpu.VMEM),
            pl.BlockSpec(memory_space=pltpu.VMEM),
        ],
        out_specs=pl.BlockSpec(memory_space=pltpu.VMEM),
        scratch_shapes=[
            pltpu.VMEM((half, n), jnp.float32),
            pltpu.VMEM((N_Y - 1, ch, n), jnp.float32),
            pltpu.VMEM((N_Y - 1, ch, n), jnp.float32),
            pltpu.SemaphoreType.DMA((N_Y - 1,)),
            pltpu.SemaphoreType.DMA((N_Y - 1,)),
            pltpu.SemaphoreType.DMA((N_Y - 1,)),
            pltpu.SemaphoreType.DMA((N_Y - 1,)),
            pltpu.SemaphoreType.DMA((N_Y,)),
            pltpu.SemaphoreType.DMA((N_Y,)),
        ],
        compiler_params=pltpu.CompilerParams(collective_id=0),
    )(dy, W)


# device time: 69608 ns/iter; 1.5065x vs baseline; 1.0912x over previous
import jax
import jax.numpy as jnp
from jax import lax
from jax.experimental import pallas as pl
from jax.experimental.pallas import tpu as pltpu

N_Y = 4
SUB = 2


def kernel(dy, W):
    m, k = dy.shape
    n = W.shape[0]
    half = m // 2
    ch = half // N_Y
    sch = ch // SUB

    def body(dy_ref, w_ref, out_ref, acc_ref, send_ref, rs_recv_ref,
             rs_send_sems, rs_recv_sems, ag_send_sems, ag_recv_sems,
             x_send_sems, x_recv_sems):
        my_x = lax.axis_index("x")
        my_y = lax.axis_index("y")
        my_z = lax.axis_index("z")
        right = (my_y + 1) % N_Y
        left = (my_y + N_Y - 1) % N_Y
        other_x = 1 - my_x
        half_off = my_x * half
        y_dev = (my_x, right, my_z)
        x_dev = (other_x, my_y, my_z)

        barrier = pltpu.get_barrier_semaphore()
        for dev in ((my_x, left, my_z), y_dev, x_dev):
            pl.semaphore_signal(
                barrier, inc=1,
                device_id=dev, device_id_type=pl.DeviceIdType.MESH,
            )
        pl.semaphore_wait(barrier, 3)

        def chunk_c(j):
            return (my_y + N_Y - j) % N_Y

        def compute_chunk(c):
            acc_ref[pl.ds(c * ch, ch), :] = lax.dot_general(
                dy_ref[pl.ds(half_off + c * ch, ch), :], w_ref[...],
                (((1,), (1,)), ((), ())),
                preferred_element_type=jnp.float32,
            )

        def rs_rdma(s, b):
            return pltpu.make_async_remote_copy(
                src_ref=send_ref.at[s, b],
                dst_ref=rs_recv_ref.at[s, b],
                send_sem=rs_send_sems.at[s, b],
                recv_sem=rs_recv_sems.at[s, b],
                device_id=y_dev, device_id_type=pl.DeviceIdType.MESH,
            )

        def ag_rdma(s, b, rows):
            return pltpu.make_async_remote_copy(
                src_ref=out_ref.at[rows, :],
                dst_ref=out_ref.at[rows, :],
                send_sem=ag_send_sems.at[s, b],
                recv_sem=ag_recv_sems.at[s, b],
                device_id=y_dev, device_id_type=pl.DeviceIdType.MESH,
            )

        def x_rdma(j, rows):
            return pltpu.make_async_remote_copy(
                src_ref=out_ref.at[rows, :],
                dst_ref=out_ref.at[rows, :],
                send_sem=x_send_sems.at[j],
                recv_sem=x_recv_sems.at[j],
                device_id=x_dev, device_id_type=pl.DeviceIdType.MESH,
            )

        drain = []

        compute_chunk(chunk_c(0))
        rs = [[None] * SUB for _ in range(N_Y - 1)]
        for b in range(SUB):
            send_ref[0, b] = acc_ref[pl.ds(chunk_c(0) * ch + b * sch, sch), :]
            rs[0][b] = rs_rdma(0, b)
            rs[0][b].start()
        compute_chunk(chunk_c(1))
        compute_chunk(chunk_c(2))
        compute_chunk(chunk_c(3))

        for s in range(1, N_Y - 1):
            c = chunk_c(s)
            for b in range(SUB):
                rs[s - 1][b].wait_recv()
                send_ref[s, b] = (
                    acc_ref[pl.ds(c * ch + b * sch, sch), :]
                    + rs_recv_ref[s - 1, b]
                )
                rs[s][b] = rs_rdma(s, b)
                rs[s][b].start()

        c_own = chunk_c(3)
        for b in range(SUB):
            rs[N_Y - 2][b].wait_recv()
            out_ref[pl.ds(half_off + c_own * ch + b * sch, sch), :] = (
                acc_ref[pl.ds(c_own * ch + b * sch, sch), :]
                + rs_recv_ref[N_Y - 2, b]
            )

        x0 = x_rdma(0, pl.ds(half_off + c_own * ch, ch))
        x0.start()
        drain.append(x0)

        ag = [[None] * SUB for _ in range(N_Y - 1)]
        for b in range(SUB):
            ag[0][b] = ag_rdma(0, b, pl.ds(half_off + c_own * ch + b * sch, sch))
            ag[0][b].start()
        for s in range(1, N_Y - 1):
            c = (my_y + 1 + N_Y - s) % N_Y
            for b in range(SUB):
                ag[s - 1][b].wait_recv()
                ag[s][b] = ag_rdma(s, b, pl.ds(half_off + c * ch + b * sch, sch))
                ag[s][b].start()
            c_fin = (my_y + N_Y - (s - 1)) % N_Y
            xs = x_rdma(s, pl.ds(half_off + c_fin * ch, ch))
            xs.start()
            drain.append(xs)
        for b in range(SUB):
            ag[N_Y - 2][b].wait_recv()
        c_fin = (my_y + N_Y - (N_Y - 2)) % N_Y
        xl = x_rdma(N_Y - 1, pl.ds(half_off + c_fin * ch, ch))
        xl.start()
        drain.append(xl)

        other_off = other_x * half
        for j in range(N_Y):
            c_j = c_own if j == 0 else (my_y + N_Y - (j - 1)) % N_Y
            x_rdma(j, pl.ds(other_off + c_j * ch, ch)).wait_recv()
        for d in drain:
            d.wait_send()
        for s in range(N_Y - 1):
            for b in range(SUB):
                rs[s][b].wait_send()
                ag[s][b].wait_send()

    return pl.pallas_call(
        body,
        out_shape=jax.ShapeDtypeStruct((m, n), jnp.float32),
        in_specs=[
            pl.BlockSpec(memory_space=pltpu.VMEM),
            pl.BlockSpec(memory_space=pltpu.VMEM),
        ],
        out_specs=pl.BlockSpec(memory_space=pltpu.VMEM),
        scratch_shapes=[
            pltpu.VMEM((half, n), jnp.float32),
            pltpu.VMEM((N_Y - 1, SUB, sch, n), jnp.float32),
            pltpu.VMEM((N_Y - 1, SUB, sch, n), jnp.float32),
            pltpu.SemaphoreType.DMA((N_Y - 1, SUB)),
            pltpu.SemaphoreType.DMA((N_Y - 1, SUB)),
            pltpu.SemaphoreType.DMA((N_Y - 1, SUB)),
            pltpu.SemaphoreType.DMA((N_Y - 1, SUB)),
            pltpu.SemaphoreType.DMA((N_Y,)),
            pltpu.SemaphoreType.DMA((N_Y,)),
        ],
        compiler_params=pltpu.CompilerParams(collective_id=0),
    )(dy, W)
